# Initial kernel scaffold; baseline (speedup 1.0000x reference)
import jax
import jax.numpy as jnp
from jax.experimental import pallas as pl


def kernel(center, pos_context, neg_context, in_embed, out_embed):
    raise NotImplementedError("write your pallas kernel here")


# speedup vs baseline: 1.5958x; 1.5958x over previous
import functools

import jax
import jax.numpy as jnp
from jax import lax
from jax.experimental import pallas as pl
from jax.experimental.pallas import tpu as pltpu
from jax.experimental.pallas import tpu_sc as plsc

NC, NS, L = 2, 16, 16
NW = NC * NS
B = 16384
D = 64
K = 5
BPW = B // NW
CH = 128
NCHUNK = BPW // CH
GRP = CH // L


def _logsig(x):
    e = jnp.exp(-jnp.abs(x))
    t = e / (2.0 + e)
    t2 = t * t
    p = 1.0 / 9.0
    for c in (1.0 / 7.0, 1.0 / 5.0, 1.0 / 3.0, 1.0):
        p = p * t2 + c
    return jnp.minimum(x, 0.0) - 2.0 * t * p


_mesh = plsc.VectorSubcoreMesh(core_axis_name="c", subcore_axis_name="s")


@functools.partial(
    pl.kernel,
    mesh=_mesh,
    compiler_params=pltpu.CompilerParams(
        needs_layout_passes=False, use_tc_tiling_on_sc=False),
    out_type=jax.ShapeDtypeStruct((NW // 8, 8 * L), jnp.float32),
    scratch_types=[
        pltpu.VMEM((BPW,), jnp.int32),
        pltpu.VMEM((BPW,), jnp.int32),
        pltpu.VMEM((BPW * K,), jnp.int32),
        pltpu.VMEM((CH, D), jnp.float32),
        pltpu.VMEM((CH, D), jnp.float32),
        pltpu.VMEM((CH, D), jnp.float32),
        pltpu.VMEM((CH, D), jnp.float32),
        pltpu.VMEM((CH * K, D), jnp.float32),
        pltpu.VMEM((CH * K, D), jnp.float32),
        pltpu.VMEM((L,), jnp.float32),
        pltpu.SemaphoreType.DMA,
        pltpu.SemaphoreType.DMA,
    ],
)
def _sc_loss(center_hbm, pos_hbm, negf_hbm, inemb_hbm, outemb_hbm,
             out_hbm, ci, pi, ni, rc0, rc1, rp0, rp1, rn0, rn1,
             accv, sem0, sem1):
    rc = (rc0, rc1)
    rp = (rp0, rp1)
    rn = (rn0, rn1)
    wid = lax.axis_index("s") * NC + lax.axis_index("c")
    base = wid * BPW

    pltpu.sync_copy(center_hbm.at[pl.ds(base, BPW)], ci)
    pltpu.sync_copy(pos_hbm.at[pl.ds(base, BPW)], pi)
    pltpu.sync_copy(negf_hbm.at[pl.ds(base * K, BPW * K)], ni)

    sems = (sem0, sem1)

    def issue(c):
        s = c % 2
        hs = [
            pltpu.async_copy(inemb_hbm.at[ci.at[pl.ds(c * CH, CH)]],
                             rc[s], sems[s]),
            pltpu.async_copy(outemb_hbm.at[pi.at[pl.ds(c * CH, CH)]],
                             rp[s], sems[s]),
        ]
        for j in range(K):
            hs.append(pltpu.async_copy(
                outemb_hbm.at[ni.at[pl.ds(c * CH * K + j * CH, CH)]],
                rn[s].at[pl.ds(j * CH, CH)], sems[s]))
        return hs

    handles = {0: issue(0)}
    acc = jnp.zeros((L,), jnp.float32)

    for c in range(NCHUNK):
        if c + 1 < NCHUNK:
            handles[c + 1] = issue(c + 1)
        for h in handles.pop(c):
            h.wait()
        s = c % 2
        rc_s, rp_s, rn_s = rc[s], rp[s], rn[s]

        def gbody(g, acc):
            rows = g * L + lax.iota(jnp.int32, L)
            rows5 = rows * K

            def dbody(d, carry):
                pos, n0, n1, n2, n3, n4 = carry
                dv = jnp.full((L,), d, jnp.int32)
                cd = plsc.load_gather(rc_s, [rows, dv])
                pd = plsc.load_gather(rp_s, [rows, dv])
                pos = pos + cd * pd
                ns = []
                for k, nk in enumerate((n0, n1, n2, n3, n4)):
                    nd = plsc.load_gather(rn_s, [rows5 + k, dv])
                    ns.append(nk + cd * nd)
                return (pos, ns[0], ns[1], ns[2], ns[3], ns[4])

            z = jnp.zeros((L,), jnp.float32)
            pos, n0, n1, n2, n3, n4 = lax.fori_loop(
                0, D, dbody, (z, z, z, z, z, z))
            tot = _logsig(pos)
            for nk in (n0, n1, n2, n3, n4):
                tot = tot + _logsig(-nk)
            return acc - tot

        acc = lax.fori_loop(0, GRP, gbody, acc)

    accv[...] = acc
    pltpu.sync_copy(accv, out_hbm.at[wid // 8, pl.ds((wid % 8) * L, L)])


def _sum_body(x_ref, o_ref):
    o_ref[...] = jnp.full((1, 1), jnp.sum(x_ref[...]) * (1.0 / B),
                          jnp.float32)


_sum = pl.pallas_call(
    _sum_body,
    out_shape=jax.ShapeDtypeStruct((1, 1), jnp.float32),
)


def kernel(center, pos_context, neg_context, in_embed, out_embed):
    center = center.astype(jnp.int32)
    pos_context = pos_context.astype(jnp.int32)
    neg_flat = neg_context.astype(jnp.int32).reshape(-1)
    partials = _sc_loss(center, pos_context, neg_flat, in_embed, out_embed)
    return _sum(partials)[0, 0]
